# baseline (speedup 1.0000x reference)
import functools

import jax
import jax.numpy as jnp
from jax import lax
from jax.experimental import pallas as pl
from jax.experimental.pallas import tpu as pltpu
from jax.experimental.pallas import tpu_sc as plsc

_NUM_WORKERS = 32


def _gather_kernel(bsz, seq, x_hbm, table_hbm, out_hbm,
                   idx_v, buf0, buf1, semg0, semg1, semw0, semw1):
    wid = lax.axis_index("s") * 2 + lax.axis_index("c")
    rows_per_w = bsz // _NUM_WORKERS
    row0 = wid * rows_per_w

    pltpu.sync_copy(x_hbm.at[pl.ds(row0, rows_per_w)], idx_v)

    def g_desc(r, buf, sem):
        return pltpu.make_async_copy(table_hbm.at[idx_v.at[r]], buf, sem)

    def w_desc(r, buf, sem):
        return pltpu.make_async_copy(buf, out_hbm.at[row0 + r], sem)

    n_pairs = rows_per_w // 2

    g_desc(0, buf0, semg0).start()

    def body(j, _):
        r0 = 2 * j
        @pl.when(j > 0)
        def _wait_w1():
            w_desc(r0 - 1, buf1, semw1).wait()

        g_desc(r0 + 1, buf1, semg1).start()
        g_desc(r0, buf0, semg0).wait()
        w_desc(r0, buf0, semw0).start()
        g_desc(r0 + 1, buf1, semg1).wait()

        @pl.when(j < n_pairs - 1)
        def _next_g0():
            w_desc(r0, buf0, semw0).wait()
            g_desc(r0 + 2, buf0, semg0).start()

        w_desc(r0 + 1, buf1, semw1).start()

        @pl.when(j == n_pairs - 1)
        def _final_waits():
            w_desc(r0, buf0, semw0).wait()
            w_desc(r0 + 1, buf1, semw1).wait()

        return 0

    lax.fori_loop(0, n_pairs, body, 0)


def kernel(x, embed):
    b, s = x.shape
    v, d = embed.shape

    mesh = plsc.VectorSubcoreMesh(core_axis_name="c", subcore_axis_name="s")

    run = pl.kernel(
        functools.partial(_gather_kernel, b, s),
        mesh=mesh,
        out_type=jax.ShapeDtypeStruct((b, s, d), jnp.float32),
        scratch_types=[
            pltpu.VMEM((b // _NUM_WORKERS, s), jnp.int32),
            pltpu.VMEM((s, d), jnp.float32),
            pltpu.VMEM((s, d), jnp.float32),
            pltpu.SemaphoreType.DMA,
            pltpu.SemaphoreType.DMA,
            pltpu.SemaphoreType.DMA,
            pltpu.SemaphoreType.DMA,
        ],
        compiler_params=pltpu.CompilerParams(use_tc_tiling_on_sc=False),
    )
    return run(x.astype(jnp.int32), embed)
